# baseline (device time: 532745 ns/iter reference)
import os

import jax
import jax.numpy as jnp
from jax import lax
from jax.experimental import pallas as pl
from jax.experimental.pallas import tpu as pltpu

N_CHUNKS = 32
W = 4
_NO_COPY = os.environ.get("DEBUG_NO_COPY", "0") == "1"
_NO_FWD = os.environ.get("DEBUG_NO_FWD", "0") == "1"


def kernel(x):
    m_per, n = x.shape
    half = m_per // 2
    K = N_CHUNKS
    chunk = half // K

    n_stage = 2 * K
    rows = m_per // n_stage

    def body(x_ref, out_ref, snd_ref, fwv_ref, stage_ref,
             snd_sems, fw_sems, in_sems, out_sems,
             d_send_sems, d_recv_sems, f_send_sems, f_recv_sems):
        my_x = lax.axis_index("x")
        my_y = lax.axis_index("y")
        y_nbr = (my_x, 1 - my_y)
        x_nbr = (1 - my_x, my_y)

        barrier_sem = pltpu.get_barrier_semaphore()
        for nbr in (y_nbr, x_nbr):
            pl.semaphore_signal(
                barrier_sem, inc=1,
                device_id=nbr, device_id_type=pl.DeviceIdType.MESH,
            )
        pl.semaphore_wait(barrier_sem, 2)

        recv_lo = (1 - my_y) * m_per + my_x * half
        fwd_recv_lo = (1 - my_y) * m_per + (1 - my_x) * half

        def dsc(lo, c):
            return pl.ds(lo + c * chunk, chunk)

        in_snd = [
            pltpu.make_async_copy(
                x_ref.at[dsc(my_x * half, c)], snd_ref.at[c % W],
                snd_sems.at[c % W])
            for c in range(K)
        ]
        rdma_d = [
            pltpu.make_async_remote_copy(
                src_ref=snd_ref.at[c % W],
                dst_ref=out_ref.at[dsc(my_y * m_per + my_x * half, c)],
                send_sem=d_send_sems.at[c], recv_sem=d_recv_sems.at[c],
                device_id=y_nbr, device_id_type=pl.DeviceIdType.MESH)
            for c in range(K)
        ]
        d_recv = [
            pltpu.make_async_remote_copy(
                src_ref=snd_ref.at[c % W],
                dst_ref=out_ref.at[dsc(recv_lo, c)],
                send_sem=d_send_sems.at[c],
                recv_sem=d_recv_sems.at[c],
                device_id=y_nbr, device_id_type=pl.DeviceIdType.MESH)
            for c in range(K)
        ]
        fw_in = [
            pltpu.make_async_copy(
                out_ref.at[dsc(recv_lo, c)], fwv_ref.at[c % W],
                fw_sems.at[c % W])
            for c in range(K)
        ]
        fwd_rdma = [
            pltpu.make_async_remote_copy(
                src_ref=fwv_ref.at[c % W],
                dst_ref=out_ref.at[dsc(recv_lo, c)],
                send_sem=f_send_sems.at[c], recv_sem=f_recv_sems.at[c],
                device_id=x_nbr, device_id_type=pl.DeviceIdType.MESH)
            for c in range(K)
        ]
        f_recv = [
            pltpu.make_async_remote_copy(
                src_ref=fwv_ref.at[c % W],
                dst_ref=out_ref.at[dsc(fwd_recv_lo, c)],
                send_sem=f_send_sems.at[c],
                recv_sem=f_recv_sems.at[c],
                device_id=x_nbr, device_id_type=pl.DeviceIdType.MESH)
            for c in range(K)
        ]
        ins = [
            pltpu.make_async_copy(
                x_ref.at[pl.ds(s * rows, rows)], stage_ref.at[s % 2],
                in_sems.at[s % 2])
            for s in range(n_stage)
        ]
        outs = [
            pltpu.make_async_copy(
                stage_ref.at[s % 2],
                out_ref.at[pl.ds(my_y * m_per + s * rows, rows)],
                out_sems.at[s % 2])
            for s in range(n_stage)
        ]

        def drive_copy_stage(s):
            if _NO_COPY:
                return
            if s + 1 < n_stage:
                if s >= 1:
                    outs[s - 1].wait()
                ins[s + 1].start()
            ins[s].wait()
            outs[s].start()

        in_snd[0].start()
        in_snd[1].start()
        if not _NO_COPY:
            ins[0].start()

        for c in range(K):
            in_snd[c].wait()
            rdma_d[c].start()
            if c + 2 < K:
                if c >= 2:
                    rdma_d[c - 2].wait_send()
                in_snd[c + 2].start()
            drive_copy_stage(2 * c)
            drive_copy_stage(2 * c + 1)
            d_recv[c].wait_recv()
            if not _NO_FWD:
                if c >= W:
                    fwd_rdma[c - W].wait_send()
                fw_in[c].start()
                if c >= 1:
                    fw_in[c - 1].wait()
                    fwd_rdma[c - 1].start()

        if not _NO_FWD:
            fw_in[K - 1].wait()
            fwd_rdma[K - 1].start()
            for c in range(K):
                f_recv[c].wait_recv()
        if not _NO_COPY:
            outs[n_stage - 2].wait()
            outs[n_stage - 1].wait()
        for c in range(max(0, K - 4), K):
            rdma_d[c].wait_send()
        if not _NO_FWD:
            for c in range(max(0, K - W), K):
                fwd_rdma[c].wait_send()

    return pl.pallas_call(
        body,
        out_shape=jax.ShapeDtypeStruct((2 * m_per, n), x.dtype),
        in_specs=[pl.BlockSpec(memory_space=pl.ANY)],
        out_specs=pl.BlockSpec(memory_space=pl.ANY),
        scratch_shapes=[
            pltpu.VMEM((W, chunk, n), x.dtype),
            pltpu.VMEM((W, chunk, n), x.dtype),
            pltpu.VMEM((2, rows, n), x.dtype),
            pltpu.SemaphoreType.DMA((W,)),
            pltpu.SemaphoreType.DMA((W,)),
            pltpu.SemaphoreType.DMA((2,)),
            pltpu.SemaphoreType.DMA((2,)),
            pltpu.SemaphoreType.DMA((K,)),
            pltpu.SemaphoreType.DMA((K,)),
            pltpu.SemaphoreType.DMA((K,)),
            pltpu.SemaphoreType.DMA((K,)),
        ],
        compiler_params=pltpu.CompilerParams(collective_id=0),
    )(x)


# device time: 478346 ns/iter; 1.1137x vs baseline; 1.1137x over previous
import os

import jax
import jax.numpy as jnp
from jax import lax
from jax.experimental import pallas as pl
from jax.experimental.pallas import tpu as pltpu

N_CHUNKS = 32
W = 4
_NO_COPY = os.environ.get("DEBUG_NO_COPY", "0") == "1"
_NO_FWD = os.environ.get("DEBUG_NO_FWD", "0") == "1"


def kernel(x):
    m_per, n = x.shape
    half = m_per // 2
    K = N_CHUNKS
    chunk = half // K

    n_stage = 2 * K
    rows = m_per // n_stage

    def body(x_ref, out_ref, snd_ref, fwv_ref, stage_ref,
             snd_sems, fw_sems, in_sems, out_sems,
             d_send_sems, d_recv_sems, f_send_sems, f_recv_sems):
        my_x = lax.axis_index("x")
        my_y = lax.axis_index("y")
        y_nbr = (my_x, 1 - my_y)
        x_nbr = (1 - my_x, my_y)

        barrier_sem = pltpu.get_barrier_semaphore()
        for nbr in (y_nbr, x_nbr):
            pl.semaphore_signal(
                barrier_sem, inc=1,
                device_id=nbr, device_id_type=pl.DeviceIdType.MESH,
            )
        pl.semaphore_wait(barrier_sem, 2)

        recv_lo = (1 - my_y) * m_per + my_x * half
        fwd_recv_lo = (1 - my_y) * m_per + (1 - my_x) * half

        def dsc(lo, c):
            return pl.ds(lo + c * chunk, chunk)

        in_snd = [
            pltpu.make_async_copy(
                x_ref.at[dsc(my_x * half, c)], snd_ref.at[c % W],
                snd_sems.at[c % W])
            for c in range(K)
        ]
        rdma_d = [
            pltpu.make_async_remote_copy(
                src_ref=snd_ref.at[c % W],
                dst_ref=out_ref.at[dsc(my_y * m_per + my_x * half, c)],
                send_sem=d_send_sems.at[c], recv_sem=d_recv_sems.at[c],
                device_id=y_nbr, device_id_type=pl.DeviceIdType.MESH)
            for c in range(K)
        ]
        d_recv = [
            pltpu.make_async_remote_copy(
                src_ref=snd_ref.at[c % W],
                dst_ref=out_ref.at[dsc(recv_lo, c)],
                send_sem=d_send_sems.at[c],
                recv_sem=d_recv_sems.at[c],
                device_id=y_nbr, device_id_type=pl.DeviceIdType.MESH)
            for c in range(K)
        ]
        fw_in = [
            pltpu.make_async_copy(
                out_ref.at[dsc(recv_lo, c)], fwv_ref.at[c % W],
                fw_sems.at[c % W])
            for c in range(K)
        ]
        fwd_rdma = [
            pltpu.make_async_remote_copy(
                src_ref=fwv_ref.at[c % W],
                dst_ref=out_ref.at[dsc(recv_lo, c)],
                send_sem=f_send_sems.at[c], recv_sem=f_recv_sems.at[c],
                device_id=x_nbr, device_id_type=pl.DeviceIdType.MESH)
            for c in range(K)
        ]
        f_recv = [
            pltpu.make_async_remote_copy(
                src_ref=fwv_ref.at[c % W],
                dst_ref=out_ref.at[dsc(fwd_recv_lo, c)],
                send_sem=f_send_sems.at[c],
                recv_sem=f_recv_sems.at[c],
                device_id=x_nbr, device_id_type=pl.DeviceIdType.MESH)
            for c in range(K)
        ]
        ins = [
            pltpu.make_async_copy(
                x_ref.at[pl.ds(s * rows, rows)], stage_ref.at[s % 2],
                in_sems.at[s % 2])
            for s in range(n_stage)
        ]
        outs = [
            pltpu.make_async_copy(
                stage_ref.at[s % 2],
                out_ref.at[pl.ds(my_y * m_per + s * rows, rows)],
                out_sems.at[s % 2])
            for s in range(n_stage)
        ]

        def drive_copy_stage(s):
            if _NO_COPY:
                return
            if s + 1 < n_stage:
                if s >= 1:
                    outs[s - 1].wait()
                ins[s + 1].start()
            ins[s].wait()
            outs[s].start()

        in_snd[0].start()
        in_snd[1].start()
        if not _NO_COPY:
            ins[0].start()

        LAG = 2
        for c in range(K + LAG):
            if c < K:
                in_snd[c].wait()
                rdma_d[c].start()
                if c + 2 < K:
                    if c >= 2:
                        rdma_d[c - 2].wait_send()
                    in_snd[c + 2].start()
                drive_copy_stage(2 * c)
                drive_copy_stage(2 * c + 1)
            r = c - LAG
            if 0 <= r:
                d_recv[r].wait_recv()
                if not _NO_FWD:
                    if r >= W:
                        fwd_rdma[r - W].wait_send()
                    fw_in[r].start()
                    if r >= 1:
                        fw_in[r - 1].wait()
                        fwd_rdma[r - 1].start()

        if not _NO_FWD:
            fw_in[K - 1].wait()
            fwd_rdma[K - 1].start()
            for c in range(K):
                f_recv[c].wait_recv()
        if not _NO_COPY:
            outs[n_stage - 2].wait()
            outs[n_stage - 1].wait()
        for c in range(max(0, K - 4), K):
            rdma_d[c].wait_send()
        if not _NO_FWD:
            for c in range(max(0, K - W), K):
                fwd_rdma[c].wait_send()

    return pl.pallas_call(
        body,
        out_shape=jax.ShapeDtypeStruct((2 * m_per, n), x.dtype),
        in_specs=[pl.BlockSpec(memory_space=pl.ANY)],
        out_specs=pl.BlockSpec(memory_space=pl.ANY),
        scratch_shapes=[
            pltpu.VMEM((W, chunk, n), x.dtype),
            pltpu.VMEM((W, chunk, n), x.dtype),
            pltpu.VMEM((2, rows, n), x.dtype),
            pltpu.SemaphoreType.DMA((W,)),
            pltpu.SemaphoreType.DMA((W,)),
            pltpu.SemaphoreType.DMA((2,)),
            pltpu.SemaphoreType.DMA((2,)),
            pltpu.SemaphoreType.DMA((K,)),
            pltpu.SemaphoreType.DMA((K,)),
            pltpu.SemaphoreType.DMA((K,)),
            pltpu.SemaphoreType.DMA((K,)),
        ],
        compiler_params=pltpu.CompilerParams(collective_id=0),
    )(x)


# device time: 465804 ns/iter; 1.1437x vs baseline; 1.0269x over previous
import os

import jax
import jax.numpy as jnp
from jax import lax
from jax.experimental import pallas as pl
from jax.experimental.pallas import tpu as pltpu

N_CHUNKS = 32
W2 = 8
LAG = 2
CL = 2
FL = 2
_NO_COPY = os.environ.get("DEBUG_NO_COPY", "0") == "1"


def kernel(x):
    m_per, n = x.shape
    half = m_per // 2
    K = N_CHUNKS
    chunk = half // K

    n_stage = 2 * K
    rows = m_per // n_stage

    def body(x_ref, out_ref, rcvd_ref, rcvf_ref, stage_ref,
             in_sems, out_sems, dr_sems, fr_sems,
             d_send_sems, d_recv_sems, f_send_sems, f_recv_sems,
             cred_d_sem, cred_f_sem):
        my_x = lax.axis_index("x")
        my_y = lax.axis_index("y")
        y_nbr = (my_x, 1 - my_y)
        x_nbr = (1 - my_x, my_y)

        barrier_sem = pltpu.get_barrier_semaphore()
        for nbr in (y_nbr, x_nbr):
            pl.semaphore_signal(
                barrier_sem, inc=1,
                device_id=nbr, device_id_type=pl.DeviceIdType.MESH,
            )
        pl.semaphore_wait(barrier_sem, 2)

        recv_lo = (1 - my_y) * m_per + my_x * half
        fwd_recv_lo = (1 - my_y) * m_per + (1 - my_x) * half

        def dsc(lo, c):
            return pl.ds(lo + c * chunk, chunk)

        rdma_d = [
            pltpu.make_async_remote_copy(
                src_ref=x_ref.at[dsc(my_x * half, c)],
                dst_ref=rcvd_ref.at[c % W2],
                send_sem=d_send_sems.at[c], recv_sem=d_recv_sems.at[c],
                device_id=y_nbr, device_id_type=pl.DeviceIdType.MESH)
            for c in range(K)
        ]
        d_recv = [
            pltpu.make_async_remote_copy(
                src_ref=x_ref.at[dsc(my_x * half, c)],
                dst_ref=rcvd_ref.at[c % W2],
                send_sem=d_send_sems.at[c],
                recv_sem=d_recv_sems.at[c],
                device_id=y_nbr, device_id_type=pl.DeviceIdType.MESH)
            for c in range(K)
        ]
        dr_out = [
            pltpu.make_async_copy(
                rcvd_ref.at[c % W2], out_ref.at[dsc(recv_lo, c)],
                dr_sems.at[c % W2])
            for c in range(K)
        ]
        fwd_rdma = [
            pltpu.make_async_remote_copy(
                src_ref=rcvd_ref.at[c % W2],
                dst_ref=rcvf_ref.at[c % W2],
                send_sem=f_send_sems.at[c], recv_sem=f_recv_sems.at[c],
                device_id=x_nbr, device_id_type=pl.DeviceIdType.MESH)
            for c in range(K)
        ]
        f_recv = [
            pltpu.make_async_remote_copy(
                src_ref=rcvd_ref.at[c % W2],
                dst_ref=rcvf_ref.at[c % W2],
                send_sem=f_send_sems.at[c],
                recv_sem=f_recv_sems.at[c],
                device_id=x_nbr, device_id_type=pl.DeviceIdType.MESH)
            for c in range(K)
        ]
        fr_out = [
            pltpu.make_async_copy(
                rcvf_ref.at[c % W2], out_ref.at[dsc(fwd_recv_lo, c)],
                fr_sems.at[c % W2])
            for c in range(K)
        ]
        ins = [
            pltpu.make_async_copy(
                x_ref.at[pl.ds(s * rows, rows)], stage_ref.at[s % 2],
                in_sems.at[s % 2])
            for s in range(n_stage)
        ]
        outs = [
            pltpu.make_async_copy(
                stage_ref.at[s % 2],
                out_ref.at[pl.ds(my_y * m_per + s * rows, rows)],
                out_sems.at[s % 2])
            for s in range(n_stage)
        ]

        def drive_copy_stage(s):
            if _NO_COPY:
                return
            if s + 1 < n_stage:
                if s >= 1:
                    outs[s - 1].wait()
                ins[s + 1].start()
            ins[s].wait()
            outs[s].start()

        def drain_direct(r2):
            dr_out[r2].wait()
            fwd_rdma[r2].wait_send()
            if r2 <= K - 1 - W2:
                pl.semaphore_signal(
                    cred_d_sem, inc=1,
                    device_id=y_nbr, device_id_type=pl.DeviceIdType.MESH)

        def drain_fwd(r4):
            fr_out[r4].wait()
            if r4 <= K - 1 - W2:
                pl.semaphore_signal(
                    cred_f_sem, inc=1,
                    device_id=x_nbr, device_id_type=pl.DeviceIdType.MESH)

        if not _NO_COPY:
            ins[0].start()

        for c in range(K + LAG):
            if c < K:
                if c >= W2:
                    pl.semaphore_wait(cred_d_sem, 1)
                rdma_d[c].start()
                drive_copy_stage(2 * c)
                drive_copy_stage(2 * c + 1)
            r = c - LAG
            if r >= 0:
                d_recv[r].wait_recv()
                dr_out[r].start()
                if r >= W2:
                    pl.semaphore_wait(cred_f_sem, 1)
                fwd_rdma[r].start()
                if r - CL >= 0:
                    drain_direct(r - CL)
                r3 = r - FL
                if r3 >= 0:
                    f_recv[r3].wait_recv()
                    fr_out[r3].start()
                    if r3 - CL >= 0:
                        drain_fwd(r3 - CL)

        for r2 in range(K - CL, K):
            drain_direct(r2)
        for r3 in range(K - FL, K):
            f_recv[r3].wait_recv()
            fr_out[r3].start()
        for r4 in range(K - FL - CL, K):
            drain_fwd(r4)
        if not _NO_COPY:
            outs[n_stage - 2].wait()
            outs[n_stage - 1].wait()
        for c in range(K):
            rdma_d[c].wait_send()

    return pl.pallas_call(
        body,
        out_shape=jax.ShapeDtypeStruct((2 * m_per, n), x.dtype),
        in_specs=[pl.BlockSpec(memory_space=pl.ANY)],
        out_specs=pl.BlockSpec(memory_space=pl.ANY),
        scratch_shapes=[
            pltpu.VMEM((W2, chunk, n), x.dtype),
            pltpu.VMEM((W2, chunk, n), x.dtype),
            pltpu.VMEM((2, rows, n), x.dtype),
            pltpu.SemaphoreType.DMA((2,)),
            pltpu.SemaphoreType.DMA((2,)),
            pltpu.SemaphoreType.DMA((W2,)),
            pltpu.SemaphoreType.DMA((W2,)),
            pltpu.SemaphoreType.DMA((K,)),
            pltpu.SemaphoreType.DMA((K,)),
            pltpu.SemaphoreType.DMA((K,)),
            pltpu.SemaphoreType.DMA((K,)),
            pltpu.SemaphoreType.REGULAR,
            pltpu.SemaphoreType.REGULAR,
        ],
        compiler_params=pltpu.CompilerParams(collective_id=0),
    )(x)


# device time: 461482 ns/iter; 1.1544x vs baseline; 1.0094x over previous
import os

import jax
import jax.numpy as jnp
from jax import lax
from jax.experimental import pallas as pl
from jax.experimental.pallas import tpu as pltpu

N_CHUNKS = 64
W2 = 8
LAG = 2
CL = 2
FL = 2
_NO_COPY = os.environ.get("DEBUG_NO_COPY", "0") == "1"


def kernel(x):
    m_per, n = x.shape
    half = m_per // 2
    K = N_CHUNKS
    chunk = half // K

    n_stage = 2 * K
    rows = m_per // n_stage

    def body(x_ref, out_ref, rcvd_ref, rcvf_ref, stage_ref,
             in_sems, out_sems, dr_sems, fr_sems,
             d_send_sems, d_recv_sems, f_send_sems, f_recv_sems,
             cred_d_sem, cred_f_sem):
        my_x = lax.axis_index("x")
        my_y = lax.axis_index("y")
        y_nbr = (my_x, 1 - my_y)
        x_nbr = (1 - my_x, my_y)

        barrier_sem = pltpu.get_barrier_semaphore()
        for nbr in (y_nbr, x_nbr):
            pl.semaphore_signal(
                barrier_sem, inc=1,
                device_id=nbr, device_id_type=pl.DeviceIdType.MESH,
            )
        pl.semaphore_wait(barrier_sem, 2)

        recv_lo = (1 - my_y) * m_per + my_x * half
        fwd_recv_lo = (1 - my_y) * m_per + (1 - my_x) * half

        def dsc(lo, c):
            return pl.ds(lo + c * chunk, chunk)

        rdma_d = [
            pltpu.make_async_remote_copy(
                src_ref=x_ref.at[dsc(my_x * half, c)],
                dst_ref=rcvd_ref.at[c % W2],
                send_sem=d_send_sems.at[c], recv_sem=d_recv_sems.at[c],
                device_id=y_nbr, device_id_type=pl.DeviceIdType.MESH)
            for c in range(K)
        ]
        d_recv = [
            pltpu.make_async_remote_copy(
                src_ref=x_ref.at[dsc(my_x * half, c)],
                dst_ref=rcvd_ref.at[c % W2],
                send_sem=d_send_sems.at[c],
                recv_sem=d_recv_sems.at[c],
                device_id=y_nbr, device_id_type=pl.DeviceIdType.MESH)
            for c in range(K)
        ]
        dr_out = [
            pltpu.make_async_copy(
                rcvd_ref.at[c % W2], out_ref.at[dsc(recv_lo, c)],
                dr_sems.at[c % W2])
            for c in range(K)
        ]
        fwd_rdma = [
            pltpu.make_async_remote_copy(
                src_ref=rcvd_ref.at[c % W2],
                dst_ref=rcvf_ref.at[c % W2],
                send_sem=f_send_sems.at[c], recv_sem=f_recv_sems.at[c],
                device_id=x_nbr, device_id_type=pl.DeviceIdType.MESH)
            for c in range(K)
        ]
        f_recv = [
            pltpu.make_async_remote_copy(
                src_ref=rcvd_ref.at[c % W2],
                dst_ref=rcvf_ref.at[c % W2],
                send_sem=f_send_sems.at[c],
                recv_sem=f_recv_sems.at[c],
                device_id=x_nbr, device_id_type=pl.DeviceIdType.MESH)
            for c in range(K)
        ]
        fr_out = [
            pltpu.make_async_copy(
                rcvf_ref.at[c % W2], out_ref.at[dsc(fwd_recv_lo, c)],
                fr_sems.at[c % W2])
            for c in range(K)
        ]
        ins = [
            pltpu.make_async_copy(
                x_ref.at[pl.ds(s * rows, rows)], stage_ref.at[s % 2],
                in_sems.at[s % 2])
            for s in range(n_stage)
        ]
        outs = [
            pltpu.make_async_copy(
                stage_ref.at[s % 2],
                out_ref.at[pl.ds(my_y * m_per + s * rows, rows)],
                out_sems.at[s % 2])
            for s in range(n_stage)
        ]

        def drive_copy_stage(s):
            if _NO_COPY:
                return
            if s + 1 < n_stage:
                if s >= 1:
                    outs[s - 1].wait()
                ins[s + 1].start()
            ins[s].wait()
            outs[s].start()

        def drain_direct(r2):
            dr_out[r2].wait()
            fwd_rdma[r2].wait_send()
            if r2 <= K - 1 - W2:
                pl.semaphore_signal(
                    cred_d_sem, inc=1,
                    device_id=y_nbr, device_id_type=pl.DeviceIdType.MESH)

        def drain_fwd(r4):
            fr_out[r4].wait()
            if r4 <= K - 1 - W2:
                pl.semaphore_signal(
                    cred_f_sem, inc=1,
                    device_id=x_nbr, device_id_type=pl.DeviceIdType.MESH)

        if not _NO_COPY:
            ins[0].start()

        for c in range(K + LAG):
            if c < K:
                if c >= W2:
                    pl.semaphore_wait(cred_d_sem, 1)
                rdma_d[c].start()
                drive_copy_stage(2 * c)
                drive_copy_stage(2 * c + 1)
            r = c - LAG
            if r >= 0:
                d_recv[r].wait_recv()
                dr_out[r].start()
                if r >= W2:
                    pl.semaphore_wait(cred_f_sem, 1)
                fwd_rdma[r].start()
                if r - CL >= 0:
                    drain_direct(r - CL)
                r3 = r - FL
                if r3 >= 0:
                    f_recv[r3].wait_recv()
                    fr_out[r3].start()
                    if r3 - CL >= 0:
                        drain_fwd(r3 - CL)

        for r2 in range(K - CL, K):
            drain_direct(r2)
        for r3 in range(K - FL, K):
            f_recv[r3].wait_recv()
            fr_out[r3].start()
        for r4 in range(K - FL - CL, K):
            drain_fwd(r4)
        if not _NO_COPY:
            outs[n_stage - 2].wait()
            outs[n_stage - 1].wait()
        for c in range(K):
            rdma_d[c].wait_send()

    return pl.pallas_call(
        body,
        out_shape=jax.ShapeDtypeStruct((2 * m_per, n), x.dtype),
        in_specs=[pl.BlockSpec(memory_space=pl.ANY)],
        out_specs=pl.BlockSpec(memory_space=pl.ANY),
        scratch_shapes=[
            pltpu.VMEM((W2, chunk, n), x.dtype),
            pltpu.VMEM((W2, chunk, n), x.dtype),
            pltpu.VMEM((2, rows, n), x.dtype),
            pltpu.SemaphoreType.DMA((2,)),
            pltpu.SemaphoreType.DMA((2,)),
            pltpu.SemaphoreType.DMA((W2,)),
            pltpu.SemaphoreType.DMA((W2,)),
            pltpu.SemaphoreType.DMA((K,)),
            pltpu.SemaphoreType.DMA((K,)),
            pltpu.SemaphoreType.DMA((K,)),
            pltpu.SemaphoreType.DMA((K,)),
            pltpu.SemaphoreType.REGULAR,
            pltpu.SemaphoreType.REGULAR,
        ],
        compiler_params=pltpu.CompilerParams(collective_id=0),
    )(x)
